# baseline (speedup 1.0000x reference)
import jax
import jax.numpy as jnp
import numpy as np
from jax.experimental import pallas as pl
from jax.experimental.pallas import tpu as pltpu

_E = 8
_TOPK = 2
_H = 1024
_I = 2048
_T = 2048
_GS = 128
_BI = 512
_NI = _I // _BI
_PK = 8
_BT = 512
_NB = _T * _TOPK // _BT + _E
_NA = _T * _TOPK


def _perm_rows(x):
    lead, tail = x.shape[:1], x.shape[2:]
    y = x.reshape(lead + (_NI, _BI // _PK, _PK) + tail)
    y = jnp.swapaxes(y, 2, 3)
    return y.reshape(lead + (_I,) + tail)


def _cumsum_shift(x, axis):
    n = x.shape[axis]
    pos = jax.lax.broadcasted_iota(jnp.int32, x.shape, axis)
    s = 1
    while s < n:
        r = jnp.roll(x, s, axis=axis)
        x = x + jnp.where(pos >= s, r, 0)
        s *= 2
    return x


def _router_body(hs_ref, gw_ref, pos1_ref, pos2_ref, p1_ref, p2_ref,
                 cnt_ref, sb_ref):
    lt = jax.lax.dot_general(
        gw_ref[...], hs_ref[...], (((1,), (1,)), ((), ())),
        preferred_element_type=jnp.float32)
    idx = jax.lax.broadcasted_iota(jnp.int32, lt.shape, 0)
    m1 = jnp.max(lt, axis=0, keepdims=True)
    a1 = jnp.min(jnp.where(lt == m1, idx, _E), axis=0, keepdims=True)
    masked = jnp.where(idx == a1, -jnp.inf, lt)
    m2 = jnp.max(masked, axis=0, keepdims=True)
    a2 = jnp.min(jnp.where(masked == m2, idx, _E), axis=0, keepdims=True)
    e2 = jnp.exp(m2 - m1)
    denom = 1.0 + e2
    p1_ref[...] = 1.0 / denom
    p2_ref[...] = e2 / denom
    oh1 = idx == a1
    oh2 = idx == a2
    c1 = _cumsum_shift(oh1.astype(jnp.int32), axis=1)
    c2 = _cumsum_shift(oh2.astype(jnp.int32), axis=1)
    count1 = c1[:, _T - 1:]
    counts = count1 + c2[:, _T - 1:]
    nblk = (counts + _BT - 1) // _BT
    start_blk = _cumsum_shift(nblk, axis=0) - nblk
    base = start_blk * _BT
    pos1_ref[...] = jnp.sum(
        jnp.where(oh1, base + c1 - 1, 0), axis=0, keepdims=True)
    pos2_ref[...] = jnp.sum(
        jnp.where(oh2, base + count1 + c2 - 1, 0), axis=0, keepdims=True)
    cnt_ref[...] = counts
    sb_ref[...] = start_blk


def _unpack_t(q, s):
    kw, n = q.shape
    srep = jnp.broadcast_to(s[:, None, :], (s.shape[0], kw // s.shape[0], n))
    srep = srep.reshape(kw, n)
    n8s = srep * -8.0
    pieces = []
    for j in range(_PK):
        u = (q >> (4 * j)) & 15
        pieces.append(u.astype(jnp.float32) * srep + n8s)
    return jnp.concatenate(pieces, axis=0).astype(jnp.bfloat16)


def _moe_body(be_ref, nv_ref, hs_ref, w1q_ref, w3q_ref, w2q_ref,
              s1_ref, s3_ref, s2_ref, b1_ref, b3_ref, b2_ref,
              ys_ref, w1s, w3s, w2s, acc):
    i = pl.program_id(0)
    b = pl.program_id(1)

    changed = jnp.logical_or(
        b == 0, be_ref[b] != be_ref[jnp.maximum(b - 1, 0)])

    @pl.when(nv_ref[b] > 0)
    def _active():
        @pl.when(changed)
        def _dequant():
            w1s[...] = _unpack_t(w1q_ref[0], s1_ref[0])
            w3s[...] = _unpack_t(w3q_ref[0], s3_ref[0])
            w2s[...] = _unpack_t(w2q_ref[0], s2_ref[0, 0])

        hs = hs_ref[...]
        nn = (((1,), (0,)), ((), ()))
        h1 = jax.lax.dot_general(hs, w1s[...], nn,
                                 preferred_element_type=jnp.float32)
        h1 = h1 + b1_ref[0]
        h1 = h1 * (1.0 / (1.0 + jnp.exp(-h1)))
        h3 = jax.lax.dot_general(hs, w3s[...], nn,
                                 preferred_element_type=jnp.float32)
        h3 = h3 + b3_ref[0]
        h13 = (h1 * h3).astype(jnp.bfloat16)
        cur = jax.lax.dot_general(h13, w2s[...], nn,
                                  preferred_element_type=jnp.float32)
        rows = pl.ds(b * _BT, _BT)

        @pl.when(i == 0)
        def _first():
            acc[rows, :] = cur + b2_ref[0]

        @pl.when(jnp.logical_and(i != 0, i != _NI - 1))
        def _rest():
            acc[rows, :] += cur

        @pl.when(i == _NI - 1)
        def _last():
            ys_ref[rows, :] = (acc[rows, :] + cur).astype(jnp.bfloat16)


@jax.jit
def kernel(hidden_states, gate_w, w1_q, w3_q, w2_q, w13_scale, w2_scale,
           w13_bias, w2_bias):
    pos1, pos2, p1, p2, counts2, sb2 = pl.pallas_call(
        _router_body,
        out_shape=(
            jax.ShapeDtypeStruct((1, _T), jnp.int32),
            jax.ShapeDtypeStruct((1, _T), jnp.int32),
            jax.ShapeDtypeStruct((1, _T), jnp.float32),
            jax.ShapeDtypeStruct((1, _T), jnp.float32),
            jax.ShapeDtypeStruct((_E, 1), jnp.int32),
            jax.ShapeDtypeStruct((_E, 1), jnp.int32),
        ),
    )(hidden_states, gate_w)

    tt = jnp.concatenate([jnp.arange(_T, dtype=jnp.int32)] * 2)
    pos = jnp.concatenate([pos1[0], pos2[0]])
    counts, sb = counts2[:, 0], sb2[:, 0]
    stp = jnp.zeros((_NB * _BT,), jnp.int32).at[pos].set(
        tt, unique_indices=True)
    bidx = jnp.arange(_NB, dtype=jnp.int32)
    block_expert = jnp.clip(
        jnp.sum((bidx[:, None] >= sb[None, :]).astype(jnp.int32), axis=1) - 1,
        0, _E - 1).astype(jnp.int32)
    beoh = (block_expert[:, None] == jnp.arange(_E, dtype=jnp.int32)[None, :]
            ).astype(jnp.int32)
    counts_b = jnp.sum(beoh * counts[None, :], axis=1)
    sb_b = jnp.sum(beoh * sb[None, :], axis=1)
    block_nvalid = jnp.clip(
        counts_b - (bidx - sb_b) * _BT, 0, _BT).astype(jnp.int32)
    ip1, ip2 = pos[:_T], pos[_T:]

    hs_p = hidden_states.reshape(_T, _H // _PK, _PK).transpose(0, 2, 1)
    hs_p = hs_p.reshape(_T, _H).astype(jnp.bfloat16)
    hs_d = jnp.take(hs_p, stp, axis=0)
    w1_qt = _perm_rows(w1_q).transpose(0, 2, 1)
    w3_qt = _perm_rows(w3_q).transpose(0, 2, 1)
    s1 = _perm_rows(w13_scale[:, :_I]).transpose(0, 2, 1)
    s3 = _perm_rows(w13_scale[:, _I:]).transpose(0, 2, 1)
    b1 = _perm_rows(w13_bias[:, :_I]).reshape(_E, 1, _I)
    b3 = _perm_rows(w13_bias[:, _I:]).reshape(_E, 1, _I)
    b2 = w2_bias.reshape(_E, 1, _H)
    w2_qt = w2_q.transpose(0, 2, 1)
    s2 = w2_scale.transpose(0, 2, 1).reshape(_E, _NI, _BI // _GS, _H)

    grid = (_NI, _NB)
    ys = pl.pallas_call(
        _moe_body,
        grid_spec=pltpu.PrefetchScalarGridSpec(
            num_scalar_prefetch=2,
            grid=grid,
            in_specs=[
                pl.BlockSpec((_BT, _H), lambda i, b, be, nv: (b, 0)),
                pl.BlockSpec((1, _H // _PK, _BI),
                             lambda i, b, be, nv: (be[b], 0, i)),
                pl.BlockSpec((1, _H // _PK, _BI),
                             lambda i, b, be, nv: (be[b], 0, i)),
                pl.BlockSpec((1, _BI // _PK, _H),
                             lambda i, b, be, nv: (be[b], i, 0)),
                pl.BlockSpec((1, _H // _GS, _BI),
                             lambda i, b, be, nv: (be[b], 0, i)),
                pl.BlockSpec((1, _H // _GS, _BI),
                             lambda i, b, be, nv: (be[b], 0, i)),
                pl.BlockSpec((1, 1, _BI // _GS, _H),
                             lambda i, b, be, nv: (be[b], i, 0, 0)),
                pl.BlockSpec((1, 1, _BI),
                             lambda i, b, be, nv: (be[b], 0, i)),
                pl.BlockSpec((1, 1, _BI),
                             lambda i, b, be, nv: (be[b], 0, i)),
                pl.BlockSpec((1, 1, _H),
                             lambda i, b, be, nv: (be[b], 0, 0)),
            ],
            out_specs=pl.BlockSpec((_NB * _BT, _H),
                                   lambda i, b, be, nv: (0, 0)),
            scratch_shapes=[
                pltpu.VMEM((_H, _BI), jnp.bfloat16),
                pltpu.VMEM((_H, _BI), jnp.bfloat16),
                pltpu.VMEM((_BI, _H), jnp.bfloat16),
                pltpu.VMEM((_NB * _BT, _H), jnp.float32),
            ],
        ),
        out_shape=jax.ShapeDtypeStruct((_NB * _BT, _H), jnp.bfloat16),
    )(block_expert, block_nvalid, hs_d, w1_qt, w3_qt, w2_qt,
      s1, s3, s2, b1, b3, b2)

    return (p1[0][:, None] * jnp.take(ys, ip1, axis=0)
            + p2[0][:, None] * jnp.take(ys, ip2, axis=0))


# speedup vs baseline: 1.1024x; 1.1024x over previous
import jax
import jax.numpy as jnp
import numpy as np
from jax.experimental import pallas as pl
from jax.experimental.pallas import tpu as pltpu

_E = 8
_TOPK = 2
_H = 1024
_I = 2048
_T = 2048
_GS = 128
_BI = 512
_NI = _I // _BI
_PK = 8
_BT = 256
_NB = _T * _TOPK // _BT + _E
_NA = _T * _TOPK


def _perm_rows(x):
    lead, tail = x.shape[:1], x.shape[2:]
    y = x.reshape(lead + (_NI, _BI // _PK, _PK) + tail)
    y = jnp.swapaxes(y, 2, 3)
    return y.reshape(lead + (_I,) + tail)


def _cumsum_shift(x, axis):
    n = x.shape[axis]
    pos = jax.lax.broadcasted_iota(jnp.int32, x.shape, axis)
    s = 1
    while s < n:
        r = jnp.roll(x, s, axis=axis)
        x = x + jnp.where(pos >= s, r, 0)
        s *= 2
    return x


def _router_body(hs_ref, gw_ref, pos1_ref, pos2_ref, p1_ref, p2_ref,
                 cnt_ref, sb_ref):
    lt = jax.lax.dot_general(
        gw_ref[...], hs_ref[...], (((1,), (1,)), ((), ())),
        preferred_element_type=jnp.float32)
    idx = jax.lax.broadcasted_iota(jnp.int32, lt.shape, 0)
    m1 = jnp.max(lt, axis=0, keepdims=True)
    a1 = jnp.min(jnp.where(lt == m1, idx, _E), axis=0, keepdims=True)
    masked = jnp.where(idx == a1, -jnp.inf, lt)
    m2 = jnp.max(masked, axis=0, keepdims=True)
    a2 = jnp.min(jnp.where(masked == m2, idx, _E), axis=0, keepdims=True)
    e2 = jnp.exp(m2 - m1)
    denom = 1.0 + e2
    p1_ref[...] = 1.0 / denom
    p2_ref[...] = e2 / denom
    oh1 = idx == a1
    oh2 = idx == a2
    c1 = _cumsum_shift(oh1.astype(jnp.int32), axis=1)
    c2 = _cumsum_shift(oh2.astype(jnp.int32), axis=1)
    count1 = c1[:, _T - 1:]
    counts = count1 + c2[:, _T - 1:]
    nblk = (counts + _BT - 1) // _BT
    start_blk = _cumsum_shift(nblk, axis=0) - nblk
    base = start_blk * _BT
    pos1_ref[...] = jnp.sum(
        jnp.where(oh1, base + c1 - 1, 0), axis=0, keepdims=True)
    pos2_ref[...] = jnp.sum(
        jnp.where(oh2, base + count1 + c2 - 1, 0), axis=0, keepdims=True)
    cnt_ref[...] = counts
    sb_ref[...] = start_blk


def _unpack_t(q, s):
    kw, n = q.shape
    srep = jnp.broadcast_to(s[:, None, :], (s.shape[0], kw // s.shape[0], n))
    srep = srep.reshape(kw, n)
    n8s = srep * -8.0
    pieces = []
    for j in range(_PK):
        u = (q >> (4 * j)) & 15
        pieces.append(u.astype(jnp.float32) * srep + n8s)
    return jnp.concatenate(pieces, axis=0).astype(jnp.bfloat16)


def _moe_body(be_ref, nv_ref, hs_ref, w1q_ref, w3q_ref, w2q_ref,
              s1_ref, s3_ref, s2_ref, b1_ref, b3_ref, b2_ref,
              ys_ref, w1s, w3s, w2s, acc):
    i = pl.program_id(0)
    b = pl.program_id(1)

    changed = jnp.logical_or(
        b == 0, be_ref[b] != be_ref[jnp.maximum(b - 1, 0)])

    @pl.when(nv_ref[b] > 0)
    def _active():
        @pl.when(changed)
        def _dequant():
            w1s[...] = _unpack_t(w1q_ref[0], s1_ref[0])
            w3s[...] = _unpack_t(w3q_ref[0], s3_ref[0])
            w2s[...] = _unpack_t(w2q_ref[0], s2_ref[0, 0])

        hs = hs_ref[...]
        nn = (((1,), (0,)), ((), ()))
        h1 = jax.lax.dot_general(hs, w1s[...], nn,
                                 preferred_element_type=jnp.float32)
        h1 = h1 + b1_ref[0]
        h1 = h1 * (1.0 / (1.0 + jnp.exp(-h1)))
        h3 = jax.lax.dot_general(hs, w3s[...], nn,
                                 preferred_element_type=jnp.float32)
        h3 = h3 + b3_ref[0]
        h13 = (h1 * h3).astype(jnp.bfloat16)
        cur = jax.lax.dot_general(h13, w2s[...], nn,
                                  preferred_element_type=jnp.float32)
        rows = pl.ds(b * _BT, _BT)

        @pl.when(i == 0)
        def _first():
            acc[rows, :] = cur + b2_ref[0]

        @pl.when(jnp.logical_and(i != 0, i != _NI - 1))
        def _rest():
            acc[rows, :] += cur

        @pl.when(i == _NI - 1)
        def _last():
            ys_ref[rows, :] = (acc[rows, :] + cur).astype(jnp.bfloat16)


@jax.jit
def kernel(hidden_states, gate_w, w1_q, w3_q, w2_q, w13_scale, w2_scale,
           w13_bias, w2_bias):
    pos1, pos2, p1, p2, counts2, sb2 = pl.pallas_call(
        _router_body,
        out_shape=(
            jax.ShapeDtypeStruct((1, _T), jnp.int32),
            jax.ShapeDtypeStruct((1, _T), jnp.int32),
            jax.ShapeDtypeStruct((1, _T), jnp.float32),
            jax.ShapeDtypeStruct((1, _T), jnp.float32),
            jax.ShapeDtypeStruct((_E, 1), jnp.int32),
            jax.ShapeDtypeStruct((_E, 1), jnp.int32),
        ),
    )(hidden_states, gate_w)

    tt = jnp.concatenate([jnp.arange(_T, dtype=jnp.int32)] * 2)
    pos = jnp.concatenate([pos1[0], pos2[0]])
    counts, sb = counts2[:, 0], sb2[:, 0]
    bidx = jnp.arange(_NB, dtype=jnp.int32)
    block_expert = jnp.clip(
        jnp.sum((bidx[:, None] >= sb[None, :]).astype(jnp.int32), axis=1) - 1,
        0, _E - 1).astype(jnp.int32)
    beoh = (block_expert[:, None] == jnp.arange(_E, dtype=jnp.int32)[None, :]
            ).astype(jnp.int32)
    counts_b = jnp.sum(beoh * counts[None, :], axis=1)
    sb_b = jnp.sum(beoh * sb[None, :], axis=1)
    block_nvalid = jnp.clip(
        counts_b - (bidx - sb_b) * _BT, 0, _BT).astype(jnp.int32)
    ip1, ip2 = pos[:_T], pos[_T:]

    hs_p = hidden_states.reshape(_T, _H // _PK, _PK).transpose(0, 2, 1)
    hs_p = hs_p.reshape(_T, _H).astype(jnp.bfloat16)
    hs_d = jnp.zeros((_NB * _BT, _H), jnp.bfloat16).at[pos].set(
        jnp.concatenate([hs_p, hs_p]), unique_indices=True)
    w1_qt = _perm_rows(w1_q).transpose(0, 2, 1)
    w3_qt = _perm_rows(w3_q).transpose(0, 2, 1)
    s1 = _perm_rows(w13_scale[:, :_I]).transpose(0, 2, 1)
    s3 = _perm_rows(w13_scale[:, _I:]).transpose(0, 2, 1)
    b1 = _perm_rows(w13_bias[:, :_I]).reshape(_E, 1, _I)
    b3 = _perm_rows(w13_bias[:, _I:]).reshape(_E, 1, _I)
    b2 = w2_bias.reshape(_E, 1, _H)
    w2_qt = w2_q.transpose(0, 2, 1)
    s2 = w2_scale.transpose(0, 2, 1).reshape(_E, _NI, _BI // _GS, _H)

    grid = (_NI, _NB)
    ys = pl.pallas_call(
        _moe_body,
        grid_spec=pltpu.PrefetchScalarGridSpec(
            num_scalar_prefetch=2,
            grid=grid,
            in_specs=[
                pl.BlockSpec((_BT, _H), lambda i, b, be, nv: (b, 0)),
                pl.BlockSpec((1, _H // _PK, _BI),
                             lambda i, b, be, nv: (be[b], 0, i)),
                pl.BlockSpec((1, _H // _PK, _BI),
                             lambda i, b, be, nv: (be[b], 0, i)),
                pl.BlockSpec((1, _BI // _PK, _H),
                             lambda i, b, be, nv: (be[b], i, 0)),
                pl.BlockSpec((1, _H // _GS, _BI),
                             lambda i, b, be, nv: (be[b], 0, i)),
                pl.BlockSpec((1, _H // _GS, _BI),
                             lambda i, b, be, nv: (be[b], 0, i)),
                pl.BlockSpec((1, 1, _BI // _GS, _H),
                             lambda i, b, be, nv: (be[b], i, 0, 0)),
                pl.BlockSpec((1, 1, _BI),
                             lambda i, b, be, nv: (be[b], 0, i)),
                pl.BlockSpec((1, 1, _BI),
                             lambda i, b, be, nv: (be[b], 0, i)),
                pl.BlockSpec((1, 1, _H),
                             lambda i, b, be, nv: (be[b], 0, 0)),
            ],
            out_specs=pl.BlockSpec((_NB * _BT, _H),
                                   lambda i, b, be, nv: (0, 0)),
            scratch_shapes=[
                pltpu.VMEM((_H, _BI), jnp.bfloat16),
                pltpu.VMEM((_H, _BI), jnp.bfloat16),
                pltpu.VMEM((_BI, _H), jnp.bfloat16),
                pltpu.VMEM((_NB * _BT, _H), jnp.float32),
            ],
        ),
        out_shape=jax.ShapeDtypeStruct((_NB * _BT, _H), jnp.bfloat16),
    )(block_expert, block_nvalid, hs_d, w1_qt, w3_qt, w2_qt,
      s1, s3, s2, b1, b3, b2)

    return (p1[0][:, None] * jnp.take(ys, ip1, axis=0)
            + p2[0][:, None] * jnp.take(ys, ip2, axis=0))
